# baseline (speedup 1.0000x reference)
import functools

import jax
import jax.numpy as jnp
from jax import lax
from jax.experimental import pallas as pl
from jax.experimental.pallas import tpu as pltpu
from jax.experimental.pallas import tpu_sc as plsc

B = 16384
DIM = 64
NEG = 512
CHUNKS = B // NEG
KCH = 128


@functools.lru_cache(maxsize=None)
def _make_sc_gather(nw: int, n_ch: int):
    b_per_w = n_ch * KCH
    mesh = plsc.VectorSubcoreMesh(core_axis_name="c", subcore_axis_name="s")
    nc = plsc.get_sparse_core_info().num_cores

    @functools.partial(
        pl.kernel,
        mesh=mesh,
        compiler_params=pltpu.CompilerParams(use_tc_tiling_on_sc=False),
        out_type=[
            jax.ShapeDtypeStruct((B, DIM), jnp.float32),
            jax.ShapeDtypeStruct((B, DIM), jnp.float32),
        ],
        scratch_types=[
            pltpu.VMEM((n_ch, KCH), jnp.int32),
            pltpu.VMEM((n_ch, KCH), jnp.int32),
            pltpu.VMEM((b_per_w, DIM), jnp.float32),
            pltpu.VMEM((b_per_w, DIM), jnp.float32),
            pltpu.SemaphoreType.DMA,
        ],
    )
    def gather_kernel(lidx_hbm, ridx_hbm, table_hbm, lhs_out, rhs_out,
                      lidx_v, ridx_v, lrows_v, rrows_v, sem):
        wid = lax.axis_index("s") * nc + lax.axis_index("c")
        base = wid * b_per_w
        pltpu.sync_copy(lidx_hbm.at[wid], lidx_v)
        pltpu.sync_copy(ridx_hbm.at[wid], ridx_v)
        copies = []
        for j in range(n_ch):
            copies.append(pltpu.async_copy(
                table_hbm.at[lidx_v.at[j]],
                lrows_v.at[pl.ds(j * KCH, KCH)], sem))
            copies.append(pltpu.async_copy(
                table_hbm.at[ridx_v.at[j]],
                rrows_v.at[pl.ds(j * KCH, KCH)], sem))
        for c in copies:
            c.wait()
        pltpu.sync_copy(lrows_v, lhs_out.at[pl.ds(base, b_per_w)])
        pltpu.sync_copy(rrows_v, rhs_out.at[pl.ds(base, b_per_w)])

    return gather_kernel


def _tc_scores(lhs, rhs, rel_vec2d):
    def body(lhs_ref, rhs_ref, rel_ref, out_ref):
        lhs_op = lhs_ref[...] * rel_ref[...]
        out_ref[0] = lax.dot_general(
            lhs_op, rhs_ref[...],
            (((1,), (1,)), ((), ())),
            preferred_element_type=jnp.float32,
            precision=lax.Precision.HIGHEST,
        )

    return pl.pallas_call(
        body,
        grid=(CHUNKS,),
        in_specs=[
            pl.BlockSpec((NEG, DIM), lambda c: (c, 0)),
            pl.BlockSpec((NEG, DIM), lambda c: (c, 0)),
            pl.BlockSpec((1, DIM), lambda c: (0, 0)),
        ],
        out_specs=pl.BlockSpec((1, NEG, NEG), lambda c: (c, 0, 0)),
        out_shape=jax.ShapeDtypeStruct((CHUNKS, NEG, NEG), jnp.float32),
    )(lhs, rhs, rel_vec2d)


def kernel(lhs_idx, rhs_idx, emb_table, rel_vec):
    info = plsc.get_sparse_core_info()
    nw = info.num_cores * info.num_subcores
    b_per_w = B // nw
    n_ch = b_per_w // KCH
    lidx3 = lhs_idx.astype(jnp.int32).reshape(nw, n_ch, KCH)
    ridx3 = rhs_idx.astype(jnp.int32).reshape(nw, n_ch, KCH)
    lhs_rows, rhs_rows = _make_sc_gather(nw, n_ch)(lidx3, ridx3, emb_table)
    return _tc_scores(lhs_rows, rhs_rows, rel_vec.reshape(1, DIM))
